# Initial kernel scaffold; baseline (speedup 1.0000x reference)
import jax
import jax.numpy as jnp
from jax.experimental import pallas as pl


def kernel(x, edge_index, edge_attr, params):
    raise NotImplementedError("write your pallas kernel here")


# speedup vs baseline: 9.7768x; 9.7768x over previous
import functools
import math

import jax
import jax.numpy as jnp
import numpy as np
from jax import lax
from jax.experimental import pallas as pl
from jax.experimental.pallas import tpu as pltpu
from jax.experimental.pallas import tpu_sc as plsc

N_NODES = 10000
N_PAD = 10240
N_EDGES = 320000
CB = 80
NW = 32
ROWS_PER_W = N_EDGES // NW
ITERS = ROWS_PER_W // CB
BN = 1000
BE = 2000


def _gather_body(idx_hbm, table_hbm, out_hbm, idx_v, rows_v, sem):
    c = lax.axis_index("c")
    s = lax.axis_index("s")
    wid = c * 16 + s

    def body(it, _):
        base = pl.multiple_of(wid * ROWS_PER_W + it * CB, CB)
        pltpu.sync_copy(idx_hbm.at[pl.ds(base, CB)], idx_v)
        pltpu.async_copy(table_hbm.at[idx_v], rows_v, sem).wait()
        pltpu.sync_copy(rows_v, out_hbm.at[pl.ds(base, CB)])
        return _

    lax.fori_loop(0, ITERS, body, None)


def _sc_gather(table, idx1d):
    n, d = table.shape
    kfn = pl.kernel(
        _gather_body,
        out_type=jax.ShapeDtypeStruct((N_EDGES, d), jnp.float32),
        mesh=plsc.VectorSubcoreMesh(core_axis_name="c", subcore_axis_name="s"),
        scratch_types=[
            pltpu.VMEM((CB,), jnp.int32),
            pltpu.VMEM((CB, d), jnp.float32),
            pltpu.SemaphoreType.DMA,
        ],
    )
    return kfn(idx1d, table)


def _scatter_body(*refs, groups, width):
    idx_hbm = refs[0]
    msgs = refs[1:1 + groups]
    zeros_hbm = refs[1 + groups]
    out_hbm = refs[2 + groups]
    idx_v, msg_v, table_sh = refs[3 + groups:]
    c = lax.axis_index("c")
    s = lax.axis_index("s")
    wid = c * 16 + s

    for g in range(groups):
        @pl.when(s == 0)
        def _zero():
            pltpu.sync_copy(zeros_hbm, table_sh)

        plsc.subcore_barrier()

        def body(it, _):
            base = pl.multiple_of(wid * ROWS_PER_W + it * CB, CB)
            pltpu.sync_copy(idx_hbm.at[pl.ds(base, CB)], idx_v)
            pltpu.sync_copy(msgs[g].at[pl.ds(base, CB)], msg_v)
            pltpu.sync_copy(msg_v, table_sh.at[idx_v], add=True)
            return _

        lax.fori_loop(0, ITERS, body, None)
        plsc.subcore_barrier()

        nch = lax.select(s < 15, 8, 5)

        def dump_body(j, _):
            off = s * 640 + j * CB
            pltpu.sync_copy(table_sh.at[pl.ds(off, CB)], msg_v)
            pltpu.sync_copy(msg_v, out_hbm.at[c, g, pl.ds(off, CB)])
            return _

        lax.fori_loop(0, nch, dump_body, None)
        plsc.subcore_barrier()


def _sc_scatter_add(msgs, idx1d, width):
    groups = len(msgs)
    zeros = jnp.zeros((N_PAD, width), jnp.float32)
    kfn = pl.kernel(
        functools.partial(_scatter_body, groups=groups, width=width),
        out_type=jax.ShapeDtypeStruct((2, groups, N_NODES, width),
                                      jnp.float32),
        mesh=plsc.VectorSubcoreMesh(core_axis_name="c", subcore_axis_name="s"),
        scratch_types=[
            pltpu.VMEM((CB,), jnp.int32),
            pltpu.VMEM((CB, width), jnp.float32),
            pltpu.VMEM_SHARED((N_PAD, width), jnp.float32),
        ],
    )
    return kfn(idx1d, *msgs, zeros)


def _proj4_kernel(x_ref, wq, bq, wk, bk, wv, bv, ws, bs, q_o, k_o, v_o, s_o):
    xb = x_ref[...]
    q_o[...] = jnp.dot(xb, wq[...], preferred_element_type=jnp.float32) + bq[...]
    k_o[...] = jnp.dot(xb, wk[...], preferred_element_type=jnp.float32) + bk[...]
    v_o[...] = jnp.dot(xb, wv[...], preferred_element_type=jnp.float32) + bv[...]
    s_o[...] = jnp.dot(xb, ws[...], preferred_element_type=jnp.float32) + bs[...]


def _proj4(x, wq, bq, wk, bk, wv, bv, ws, bs):
    n, din = x.shape
    hc = wq.shape[1]
    full = lambda a: pl.BlockSpec(a.shape, lambda i: (0,) * a.ndim)
    row = lambda w: pl.BlockSpec((BN, w), lambda i: (i, 0))
    outs = [jax.ShapeDtypeStruct((n, hc), jnp.float32)] * 4
    return pl.pallas_call(
        _proj4_kernel,
        grid=(n // BN,),
        in_specs=[row(din)] + [full(a) for a in (wq, bq, wk, bk, wv, bv, ws, bs)],
        out_specs=[row(hc)] * 4,
        out_shape=outs,
    )(x, wq, bq, wk, bk, wv, bv, ws, bs)


def _alpha_kernel(qd, ks, ea, we, msum, alpha_o, gmax_o, *, scale):
    e = jnp.dot(ea[...], we[...], preferred_element_type=jnp.float32)
    s = qd[...] * (ks[...] + e)
    alpha = jnp.dot(s, msum[...], preferred_element_type=jnp.float32) * scale
    alpha_o[...] = alpha
    bm = jnp.broadcast_to(jnp.max(alpha, axis=0, keepdims=True), (8, 8))
    i = pl.program_id(0)

    @pl.when(i == 0)
    def _init():
        gmax_o[...] = bm

    @pl.when(i != 0)
    def _acc():
        gmax_o[...] = jnp.maximum(gmax_o[...], bm)


def _alpha(qd, ks, ea, we, msum, c_dim):
    e_tot, hc = qd.shape
    full = lambda a: pl.BlockSpec(a.shape, lambda i: (0,) * a.ndim)
    return pl.pallas_call(
        functools.partial(_alpha_kernel, scale=1.0 / math.sqrt(c_dim)),
        grid=(e_tot // BE,),
        in_specs=[
            pl.BlockSpec((BE, hc), lambda i: (i, 0)),
            pl.BlockSpec((BE, hc), lambda i: (i, 0)),
            pl.BlockSpec((BE, 22), lambda i: (i, 0)),
            full(we), full(msum),
        ],
        out_specs=[
            pl.BlockSpec((BE, 8), lambda i: (i, 0)),
            pl.BlockSpec((8, 8), lambda i: (0, 0)),
        ],
        out_shape=[
            jax.ShapeDtypeStruct((e_tot, 8), jnp.float32),
            jax.ShapeDtypeStruct((8, 8), jnp.float32),
        ],
    )(qd, ks, ea, we, msum)


def _msg_kernel(alpha, gmax, vs, ea, we, mbc, *outs, heads, width):
    ex = jnp.exp(alpha[...] - gmax[0:1, :])
    e = jnp.dot(ea[...], we[...], preferred_element_type=jnp.float32)
    exb = jnp.dot(ex, mbc[...], preferred_element_type=jnp.float32)
    msg = exb * (vs[...] + e)
    for h in range(heads):
        outs[h][...] = msg[:, h * width:(h + 1) * width]
    outs[heads][...] = jnp.concatenate(
        [ex, jnp.zeros((ex.shape[0], width - 8), ex.dtype)], axis=1)


def _msg(alpha, gmax, vs, ea, we, mbc, heads, width):
    e_tot, hc = vs.shape
    full = lambda a: pl.BlockSpec(a.shape, lambda i: (0,) * a.ndim)
    res = pl.pallas_call(
        functools.partial(_msg_kernel, heads=heads, width=width),
        grid=(e_tot // BE,),
        in_specs=[
            pl.BlockSpec((BE, 8), lambda i: (i, 0)),
            full(gmax),
            pl.BlockSpec((BE, hc), lambda i: (i, 0)),
            pl.BlockSpec((BE, 22), lambda i: (i, 0)),
            full(we), full(mbc),
        ],
        out_specs=[pl.BlockSpec((BE, width), lambda i: (i, 0))] * (heads + 1),
        out_shape=[jax.ShapeDtypeStruct((e_tot, width), jnp.float32)]
        * (heads + 1),
    )(alpha, gmax, vs, ea, we, mbc)
    return res


def _assemble_kernel(aggp, skip, out, *, heads, width):
    agg = aggp[0] + aggp[1]
    den = agg[heads][:, 0:8]
    cols = []
    for h in range(heads):
        cols.append(agg[h] / (den[:, h:h + 1] + 1e-16))
    out[...] = jnp.maximum(jnp.concatenate(cols, axis=1) + skip[...], 0.0)


def _assemble(aggp, skip, heads, width):
    n, hc = skip.shape
    return pl.pallas_call(
        functools.partial(_assemble_kernel, heads=heads, width=width),
        grid=(n // BN,),
        in_specs=[
            pl.BlockSpec((2, heads + 1, BN, width), lambda i: (0, 0, i, 0)),
            pl.BlockSpec((BN, hc), lambda i: (i, 0)),
        ],
        out_specs=pl.BlockSpec((BN, hc), lambda i: (i, 0)),
        out_shape=jax.ShapeDtypeStruct((n, hc), jnp.float32),
    )(aggp, skip)


def _final_kernel(x1, x2, w1, w2, bl, out):
    acc = jnp.sum(x1[...] * w1[...], axis=1, keepdims=True)
    acc = acc + jnp.sum(x2[...] * w2[...], axis=1, keepdims=True)
    out[...] = jax.nn.sigmoid(acc + bl[...])


def _final(x1, x2, wl, bl):
    n = x1.shape[0]
    d1 = x1.shape[1]
    w1 = wl[:d1, 0].reshape(1, d1)
    w2 = wl[d1:, 0].reshape(1, wl.shape[0] - d1)
    blr = bl.reshape(1, 1)
    full = lambda a: pl.BlockSpec(a.shape, lambda i: (0,) * a.ndim)
    return pl.pallas_call(
        _final_kernel,
        grid=(n // BN,),
        in_specs=[
            pl.BlockSpec((BN, d1), lambda i: (i, 0)),
            pl.BlockSpec((BN, x2.shape[1]), lambda i: (i, 0)),
            full(w1), full(w2), full(blr),
        ],
        out_specs=pl.BlockSpec((BN, 1), lambda i: (i, 0)),
        out_shape=jax.ShapeDtypeStruct((n, 1), jnp.float32),
    )(x1, x2, w1, w2, blr)


def _conv_layer(x, src1d, dst1d, ea, wq, bq, wk, bk, wv, bv, we, ws, bs,
                heads, c_dim):
    hc = heads * c_dim
    msum = jnp.asarray(np.kron(np.eye(heads), np.ones((c_dim, 1))),
                       jnp.float32)
    mbc = jnp.asarray(np.kron(np.eye(heads), np.ones((1, c_dim))),
                      jnp.float32)
    q, k, v, sk = _proj4(x, wq, bq.reshape(1, hc), wk, bk.reshape(1, hc),
                         wv, bv.reshape(1, hc), ws, bs.reshape(1, hc))
    qd = _sc_gather(q, dst1d)
    ks = _sc_gather(k, src1d)
    vs = _sc_gather(v, src1d)
    alpha, gmax = _alpha(qd, ks, ea, we, msum, c_dim)
    msgs = _msg(alpha, gmax, vs, ea, we, mbc, heads, c_dim)
    aggp = _sc_scatter_add(list(msgs), dst1d, c_dim)
    return _assemble(aggp, sk, heads, c_dim)


def kernel(x, edge_index, edge_attr, params):
    p = params
    src1d = edge_index[0].astype(jnp.int32)
    dst1d = edge_index[1].astype(jnp.int32)
    x1 = _conv_layer(x, src1d, dst1d, edge_attr,
                     p['Wq1'], p['bq1'], p['Wk1'], p['bk1'], p['Wv1'],
                     p['bv1'], p['We1'], p['Ws1'], p['bs1'], 8, 32)
    x2 = _conv_layer(x1, src1d, dst1d, edge_attr,
                     p['Wq2'], p['bq2'], p['Wk2'], p['bk2'], p['Wv2'],
                     p['bv2'], p['We2'], p['Ws2'], p['bs2'], 8, 64)
    return _final(x1, x2, p['Wl'], p['bl'])


# speedup vs baseline: 11.0149x; 1.1266x over previous
import functools
import math

import jax
import jax.numpy as jnp
import numpy as np
from jax import lax
from jax.experimental import pallas as pl
from jax.experimental.pallas import tpu as pltpu
from jax.experimental.pallas import tpu_sc as plsc

N_NODES = 10000
N_PAD = 10240
N_EDGES = 320000
CB = 80
NW = 32
ROWS_PER_W = N_EDGES // NW
ITERS = ROWS_PER_W // CB
BN = 1000
BE = 2000


def _gather_body(idx_hbm, table_hbm, out_hbm, idx_v, rows_v, sem):
    c = lax.axis_index("c")
    s = lax.axis_index("s")
    wid = c * 16 + s

    def body(it, _):
        base = pl.multiple_of(wid * ROWS_PER_W + it * CB, CB)
        pltpu.sync_copy(idx_hbm.at[pl.ds(base, CB)], idx_v)
        pltpu.async_copy(table_hbm.at[idx_v], rows_v, sem).wait()
        pltpu.sync_copy(rows_v, out_hbm.at[pl.ds(base, CB)])
        return _

    lax.fori_loop(0, ITERS, body, None)


def _sc_gather(table, idx1d):
    n, d = table.shape
    kfn = pl.kernel(
        _gather_body,
        out_type=jax.ShapeDtypeStruct((N_EDGES, d), jnp.float32),
        mesh=plsc.VectorSubcoreMesh(core_axis_name="c", subcore_axis_name="s"),
        scratch_types=[
            pltpu.VMEM((CB,), jnp.int32),
            pltpu.VMEM((CB, d), jnp.float32),
            pltpu.SemaphoreType.DMA,
        ],
    )
    return kfn(idx1d, table)


def _scatter_body(*refs, groups, width):
    idx_hbm = refs[0]
    msgs = refs[1:1 + groups]
    zeros_hbm = refs[1 + groups]
    out_hbm = refs[2 + groups]
    idx_v, msg_v, table_sh = refs[3 + groups:]
    c = lax.axis_index("c")
    s = lax.axis_index("s")
    wid = c * 16 + s

    for g in range(groups):
        @pl.when(s == 0)
        def _zero():
            pltpu.sync_copy(zeros_hbm, table_sh)

        plsc.subcore_barrier()

        def body(it, _):
            base = pl.multiple_of(wid * ROWS_PER_W + it * CB, CB)
            pltpu.sync_copy(idx_hbm.at[pl.ds(base, CB)], idx_v)
            pltpu.sync_copy(msgs[g].at[pl.ds(base, CB)], msg_v)
            pltpu.sync_copy(msg_v, table_sh.at[idx_v], add=True)
            return _

        lax.fori_loop(0, ITERS, body, None)
        plsc.subcore_barrier()

        nch = lax.select(s < 15, 8, 5)

        def dump_body(j, _):
            off = s * 640 + j * CB
            pltpu.sync_copy(table_sh.at[pl.ds(off, CB)], msg_v)
            pltpu.sync_copy(msg_v, out_hbm.at[c, g, pl.ds(off, CB)])
            return _

        lax.fori_loop(0, nch, dump_body, None)
        plsc.subcore_barrier()


def _sc_scatter_add(msgs, idx1d, width):
    groups = len(msgs)
    zeros = jnp.zeros((N_PAD, width), jnp.float32)
    kfn = pl.kernel(
        functools.partial(_scatter_body, groups=groups, width=width),
        out_type=jax.ShapeDtypeStruct((2, groups, N_NODES, width),
                                      jnp.float32),
        mesh=plsc.VectorSubcoreMesh(core_axis_name="c", subcore_axis_name="s"),
        scratch_types=[
            pltpu.VMEM((CB,), jnp.int32),
            pltpu.VMEM((CB, width), jnp.float32),
            pltpu.VMEM_SHARED((N_PAD, width), jnp.float32),
        ],
    )
    return kfn(idx1d, *msgs, zeros)


def _proj4_kernel(x_ref, wq, bq, wk, bk, wv, bv, ws, bs, q_o, kv_o, s_o, *, hc):
    xb = x_ref[...]
    q_o[...] = jnp.dot(xb, wq[...], preferred_element_type=jnp.float32) + bq[...]
    kv_o[:, :hc] = jnp.dot(xb, wk[...], preferred_element_type=jnp.float32) + bk[...]
    kv_o[:, hc:] = jnp.dot(xb, wv[...], preferred_element_type=jnp.float32) + bv[...]
    s_o[...] = jnp.dot(xb, ws[...], preferred_element_type=jnp.float32) + bs[...]


def _proj4(x, wq, bq, wk, bk, wv, bv, ws, bs):
    n, din = x.shape
    hc = wq.shape[1]
    full = lambda a: pl.BlockSpec(a.shape, lambda i: (0,) * a.ndim)
    row = lambda w: pl.BlockSpec((BN, w), lambda i: (i, 0))
    outs = [jax.ShapeDtypeStruct((n, hc), jnp.float32),
            jax.ShapeDtypeStruct((n, 2 * hc), jnp.float32),
            jax.ShapeDtypeStruct((n, hc), jnp.float32)]
    return pl.pallas_call(
        functools.partial(_proj4_kernel, hc=hc),
        grid=(n // BN,),
        in_specs=[row(din)] + [full(a) for a in (wq, bq, wk, bk, wv, bv, ws, bs)],
        out_specs=[row(hc), row(2 * hc), row(hc)],
        out_shape=outs,
    )(x, wq, bq, wk, bk, wv, bv, ws, bs)


def _alpha_kernel(qd, ks, ea, we, msum, alpha_o, gmax_o, *, scale):
    e = jnp.dot(ea[...], we[...], preferred_element_type=jnp.float32)
    s = qd[...] * (ks[...] + e)
    alpha = jnp.dot(s, msum[...], preferred_element_type=jnp.float32) * scale
    alpha_o[...] = alpha
    bm = jnp.broadcast_to(jnp.max(alpha, axis=0, keepdims=True), (8, 8))
    i = pl.program_id(0)

    @pl.when(i == 0)
    def _init():
        gmax_o[...] = bm

    @pl.when(i != 0)
    def _acc():
        gmax_o[...] = jnp.maximum(gmax_o[...], bm)


def _alpha(qd, karr, kcol, ea, we, msum, c_dim):
    e_tot, hc = qd.shape
    full = lambda a: pl.BlockSpec(a.shape, lambda i: (0,) * a.ndim)
    return pl.pallas_call(
        functools.partial(_alpha_kernel, scale=1.0 / math.sqrt(c_dim)),
        grid=(e_tot // BE,),
        in_specs=[
            pl.BlockSpec((BE, hc), lambda i: (i, 0)),
            pl.BlockSpec((BE, hc), lambda i: (i, kcol)),
            pl.BlockSpec((BE, 22), lambda i: (i, 0)),
            full(we), full(msum),
        ],
        out_specs=[
            pl.BlockSpec((BE, 8), lambda i: (i, 0)),
            pl.BlockSpec((8, 8), lambda i: (0, 0)),
        ],
        out_shape=[
            jax.ShapeDtypeStruct((e_tot, 8), jnp.float32),
            jax.ShapeDtypeStruct((8, 8), jnp.float32),
        ],
    )(qd, karr, ea, we, msum)


def _msg_kernel(alpha, gmax, kv, ea, we, mbc, *outs, heads, width, pack):
    ex = jnp.exp(alpha[...] - gmax[0:1, :])
    e = jnp.dot(ea[...], we[...], preferred_element_type=jnp.float32)
    exb = jnp.dot(ex, mbc[...], preferred_element_type=jnp.float32)
    msg = exb * (kv[...] + e)
    n_groups = heads // pack
    gw = pack * width
    for g in range(n_groups):
        outs[g][...] = msg[:, g * gw:(g + 1) * gw]
    outs[n_groups][...] = jnp.concatenate(
        [ex, jnp.zeros((ex.shape[0], 24), ex.dtype)], axis=1)


def _msg(alpha, gmax, varr, vcol, ea, we, mbc, heads, width, pack):
    e_tot = varr.shape[0]
    hc = heads * width
    n_groups = heads // pack
    full = lambda a: pl.BlockSpec(a.shape, lambda i: (0,) * a.ndim)
    res = pl.pallas_call(
        functools.partial(_msg_kernel, heads=heads, width=width, pack=pack),
        grid=(e_tot // BE,),
        in_specs=[
            pl.BlockSpec((BE, 8), lambda i: (i, 0)),
            full(gmax),
            pl.BlockSpec((BE, hc), lambda i: (i, vcol)),
            pl.BlockSpec((BE, 22), lambda i: (i, 0)),
            full(we), full(mbc),
        ],
        out_specs=[pl.BlockSpec((BE, pack * width), lambda i: (i, 0))]
        * n_groups + [pl.BlockSpec((BE, 32), lambda i: (i, 0))],
        out_shape=[jax.ShapeDtypeStruct((e_tot, pack * width), jnp.float32)]
        * n_groups + [jax.ShapeDtypeStruct((e_tot, 32), jnp.float32)],
    )(alpha, gmax, varr, ea, we, mbc)
    return res


def _assemble_kernel(aggh, aggd, skip, out, *, heads, width, pack):
    agg = aggh[0] + aggh[1]
    den = (aggd[0, 0] + aggd[1, 0])[:, 0:8]
    cols = []
    for h in range(heads):
        g, j = divmod(h, pack)
        cols.append(agg[g][:, j * width:(j + 1) * width]
                    / (den[:, h:h + 1] + 1e-16))
    out[...] = jnp.maximum(jnp.concatenate(cols, axis=1) + skip[...], 0.0)


def _assemble(aggh, aggd, skip, heads, width, pack):
    n, hc = skip.shape
    n_groups = heads // pack
    return pl.pallas_call(
        functools.partial(_assemble_kernel, heads=heads, width=width,
                          pack=pack),
        grid=(n // BN,),
        in_specs=[
            pl.BlockSpec((2, n_groups, BN, pack * width),
                         lambda i: (0, 0, i, 0)),
            pl.BlockSpec((2, 1, BN, 32), lambda i: (0, 0, i, 0)),
            pl.BlockSpec((BN, hc), lambda i: (i, 0)),
        ],
        out_specs=pl.BlockSpec((BN, hc), lambda i: (i, 0)),
        out_shape=jax.ShapeDtypeStruct((n, hc), jnp.float32),
    )(aggh, aggd, skip)


def _final_kernel(x1, x2, w1, w2, bl, out):
    acc = jnp.sum(x1[...] * w1[...], axis=1, keepdims=True)
    acc = acc + jnp.sum(x2[...] * w2[...], axis=1, keepdims=True)
    out[...] = jax.nn.sigmoid(acc + bl[...])


def _final(x1, x2, wl, bl):
    n = x1.shape[0]
    d1 = x1.shape[1]
    w1 = wl[:d1, 0].reshape(1, d1)
    w2 = wl[d1:, 0].reshape(1, wl.shape[0] - d1)
    blr = bl.reshape(1, 1)
    full = lambda a: pl.BlockSpec(a.shape, lambda i: (0,) * a.ndim)
    return pl.pallas_call(
        _final_kernel,
        grid=(n // BN,),
        in_specs=[
            pl.BlockSpec((BN, d1), lambda i: (i, 0)),
            pl.BlockSpec((BN, x2.shape[1]), lambda i: (i, 0)),
            full(w1), full(w2), full(blr),
        ],
        out_specs=pl.BlockSpec((BN, 1), lambda i: (i, 0)),
        out_shape=jax.ShapeDtypeStruct((n, 1), jnp.float32),
    )(x1, x2, w1, w2, blr)


def _conv_layer(x, src1d, dst1d, ea, wq, bq, wk, bk, wv, bv, we, ws, bs,
                heads, c_dim):
    hc = heads * c_dim
    msum = jnp.asarray(np.kron(np.eye(heads), np.ones((c_dim, 1))),
                       jnp.float32)
    mbc = jnp.asarray(np.kron(np.eye(heads), np.ones((1, c_dim))),
                      jnp.float32)
    pack = 64 // c_dim
    q, kv, sk = _proj4(x, wq, bq.reshape(1, hc), wk, bk.reshape(1, hc),
                       wv, bv.reshape(1, hc), ws, bs.reshape(1, hc))
    qd = _sc_gather(q, dst1d)
    if hc <= 256:
        kvs = _sc_gather(kv, src1d)
        karr, kcol = kvs, 0
        varr, vcol = kvs, 1
    else:
        karr = _sc_gather(kv[:, :hc], src1d)
        varr = _sc_gather(kv[:, hc:], src1d)
        kcol = vcol = 0
    alpha, gmax = _alpha(qd, karr, kcol, ea, we, msum, c_dim)
    msgs = _msg(alpha, gmax, varr, vcol, ea, we, mbc, heads, c_dim, pack)
    aggh = _sc_scatter_add(list(msgs[:-1]), dst1d, pack * c_dim)
    aggd = _sc_scatter_add([msgs[-1]], dst1d, 32)
    return _assemble(aggh, aggd, sk, heads, c_dim, pack)


def kernel(x, edge_index, edge_attr, params):
    p = params
    src1d = edge_index[0].astype(jnp.int32)
    dst1d = edge_index[1].astype(jnp.int32)
    x1 = _conv_layer(x, src1d, dst1d, edge_attr,
                     p['Wq1'], p['bq1'], p['Wk1'], p['bk1'], p['Wv1'],
                     p['bv1'], p['We1'], p['Ws1'], p['bs1'], 8, 32)
    x2 = _conv_layer(x1, src1d, dst1d, edge_attr,
                     p['Wq2'], p['bq2'], p['Wk2'], p['bk2'], p['Wv2'],
                     p['bv2'], p['We2'], p['Ws2'], p['bs2'], 8, 64)
    return _final(x1, x2, p['Wl'], p['bl'])
